# baseline (speedup 1.0000x reference)
import functools

import jax
import jax.numpy as jnp
from jax import lax
from jax.experimental import pallas as pl
from jax.experimental.pallas import tpu as pltpu
from jax.experimental.pallas import tpu_sc as plsc

BATCH = 4096
HIST = 200
EMBED_DIM = 64
VOCAB = 1000000
B_TOTAL = BATCH * HIST

_info = plsc.get_sparse_core_info()
NUM_CORES = _info.num_cores
NUM_SUBCORES = _info.num_subcores
NW = NUM_CORES * NUM_SUBCORES
STRIPE = BATCH // NW

TBLK = 8192


def _transpose_body(wt_ref, out_ref):
    eye = jnp.eye(EMBED_DIM, dtype=jnp.float32)
    out_ref[...] = jax.lax.dot_general(
        wt_ref[...], eye, (((0,), (0,)), ((), ())),
        preferred_element_type=jnp.float32,
    )


def _transpose_weight(weight):
    wt = weight.T
    vocab = wt.shape[1]
    grid = (vocab + TBLK - 1) // TBLK
    return pl.pallas_call(
        _transpose_body,
        grid=(grid,),
        in_specs=[pl.BlockSpec((EMBED_DIM, TBLK), lambda i: (0, i))],
        out_specs=pl.BlockSpec((TBLK, EMBED_DIM), lambda i: (i, 0)),
        out_shape=jax.ShapeDtypeStruct((vocab, EMBED_DIM), jnp.float32),
    )(wt)


def _gather_body(wl_hbm, idx_hbm, out_hbm, idx_v, rows, blk, gsem, wsem):
    c = lax.axis_index("c")
    s = lax.axis_index("s")
    w = s * NUM_CORES + c

    pltpu.sync_copy(idx_hbm.at[:, w, :], idx_v)

    def gather(u, i):
        pltpu.make_async_copy(
            wl_hbm.at[idx_v.at[u, :]], rows[i], gsem.at[i]
        ).start()

    def gather_wait(i):
        pltpu.make_async_copy(
            wl_hbm.at[idx_v.at[0, :]], rows[i], gsem.at[i]
        ).wait()

    def write(u, i):
        pltpu.make_async_copy(
            blk[i], out_hbm.at[u, :, w], wsem.at[i]
        ).start()

    def write_wait(i):
        pltpu.make_async_copy(
            blk[i], out_hbm.at[0, :, w], wsem.at[i]
        ).wait()

    iota = lax.broadcasted_iota(jnp.int32, (16,), 0)
    rowv = [iota + 16 * j for j in range(STRIPE // 16)]

    DG = 4

    def _t_loads(i, d0):
        vs = []
        for d in range(d0, d0 + DG):
            col = jnp.full((16,), d, jnp.int32)
            for j in range(STRIPE // 16):
                vs.append(plsc.load_gather(rows[i], [rowv[j], col]))
        return vs

    def _t_stores(i, d0, vs):
        for k, d in enumerate(range(d0, d0 + DG)):
            for j in range(STRIPE // 16):
                blk[i][d // 8, d % 8, pl.ds(16 * j, 16)] = vs[k * 8 + j]

    def transpose(i):
        prev = None
        for d0 in range(0, EMBED_DIM, DG):
            vs = _t_loads(i, d0)
            if prev is not None:
                _t_stores(i, d0 - DG, prev)
            prev = vs
        _t_stores(i, EMBED_DIM - DG, prev)

    for i in range(2):
        gather(i, i)
        write(i, i)

    def body(o, carry):
        for t in range(2):
            u = o * 2 + t
            gather_wait(t)
            write_wait(t)
            transpose(t)
            write(u, t)
            @pl.when(u < HIST - 2)
            def _():
                gather(u + 2, t)
        return carry

    lax.fori_loop(0, HIST // 2, body, 0)

    for t in range(2):
        write_wait(t)


@jax.jit
def _embedding_lookup(idx3, weight):
    wl = _transpose_weight(weight)
    mesh = plsc.VectorSubcoreMesh(core_axis_name="c", subcore_axis_name="s")
    fn = pl.kernel(
        _gather_body,
        out_type=jax.ShapeDtypeStruct((HIST, 8, NW, 8, STRIPE), jnp.float32),
        mesh=mesh,
        scratch_types=[
            pltpu.VMEM((HIST, STRIPE), jnp.int32),
            [pltpu.VMEM((STRIPE, EMBED_DIM), jnp.float32) for _ in range(2)],
            [pltpu.VMEM((8, 8, STRIPE), jnp.float32) for _ in range(2)],
            pltpu.SemaphoreType.DMA((2,)),
            pltpu.SemaphoreType.DMA((2,)),
        ],
        compiler_params=pltpu.CompilerParams(
            use_tc_tiling_on_sc=False, needs_layout_passes=False
        ),
    )
    return fn(wl, idx3)


def kernel(input, weight):
    idx3 = input.T.reshape(HIST, NW, STRIPE).astype(jnp.int32)
    out5 = _embedding_lookup(idx3, weight)
    return out5.transpose(2, 4, 0, 1, 3).reshape(BATCH, HIST, EMBED_DIM)


# speedup vs baseline: 1.3366x; 1.3366x over previous
import functools

import jax
import jax.numpy as jnp
from jax import lax
from jax.experimental import pallas as pl
from jax.experimental.pallas import tpu as pltpu
from jax.experimental.pallas import tpu_sc as plsc

BATCH = 4096
HIST = 200
EMBED_DIM = 64
B_TOTAL = BATCH * HIST

_info = plsc.get_sparse_core_info()
NUM_CORES = _info.num_cores
NUM_SUBCORES = _info.num_subcores
NW = NUM_CORES * NUM_SUBCORES
B_PER_W = B_TOTAL // NW

K = 4
CHUNK = 400
NCHUNK = B_PER_W // CHUNK
NGROUP = NCHUNK // K


def _gather_body(table_hbm, idx_hbm, out_hbm, idx_v, rows, gsem, wsem):
    c = lax.axis_index("c")
    s = lax.axis_index("s")
    wid = s * NUM_CORES + c
    base = wid * B_PER_W
    pltpu.sync_copy(idx_hbm.at[pl.ds(base, B_PER_W)], idx_v)

    def gather(g, i):
        off = pl.multiple_of(g * CHUNK, CHUNK)
        pltpu.make_async_copy(
            table_hbm.at[idx_v.at[pl.ds(off, CHUNK)]], rows[i], gsem.at[i]
        ).start()

    def write(g, i):
        off = pl.multiple_of(g * CHUNK, CHUNK)
        pltpu.make_async_copy(
            rows[i], out_hbm.at[pl.ds(base + off, CHUNK)], wsem.at[i]
        ).start()

    def gather_wait(i):
        pltpu.make_async_copy(
            table_hbm.at[idx_v.at[pl.ds(0, CHUNK)]], rows[i], gsem.at[i]
        ).wait()

    def write_wait(i):
        pltpu.make_async_copy(
            rows[i], out_hbm.at[pl.ds(base, CHUNK)], wsem.at[i]
        ).wait()

    for i in range(K):
        gather(i, i)

    def body(o, carry):
        g0 = o * K
        for i in range(K):
            gather_wait(i)
            write(g0 + i, i)
        for i in range(K):
            write_wait(i)
            gather(g0 + K + i, i)
        return carry

    lax.fori_loop(0, NGROUP - 1, body, 0)

    g0 = (NGROUP - 1) * K
    for i in range(K):
        gather_wait(i)
        write(g0 + i, i)
    for i in range(K):
        write_wait(i)


@jax.jit
def _embedding_lookup(idx_flat, weight):
    mesh = plsc.VectorSubcoreMesh(core_axis_name="c", subcore_axis_name="s")
    fn = pl.kernel(
        _gather_body,
        out_type=jax.ShapeDtypeStruct((B_TOTAL, EMBED_DIM), jnp.float32),
        mesh=mesh,
        scratch_types=[
            pltpu.VMEM((B_PER_W,), jnp.int32),
            [pltpu.VMEM((CHUNK, EMBED_DIM), jnp.float32) for _ in range(K)],
            pltpu.SemaphoreType.DMA((K,)),
            pltpu.SemaphoreType.DMA((K,)),
        ],
        compiler_params=pltpu.CompilerParams(use_tc_tiling_on_sc=False),
    )
    return fn(weight, idx_flat)


def kernel(input, weight):
    idx_flat = input.reshape(-1).astype(jnp.int32)
    out = _embedding_lookup(idx_flat, weight)
    return out.reshape(BATCH, HIST, EMBED_DIM)
